# baseline (speedup 1.0000x reference)
import functools

import jax
import jax.numpy as jnp
import numpy as np
from jax import lax
from jax.experimental import pallas as pl
from jax.experimental.pallas import tpu as pltpu
from jax.experimental.pallas import tpu_sc as plsc

_B = 4096
_S = 50
_D = 64

_NC = 2
_NS = 16
_NW = _NC * _NS
_L = 16

_SPW = _B // _NW
_PB = 1024
_NPB = 489
_VP = _NPB * _PB


def _pos_rep_body(radius_ref, period_ref, phase_ref, out_ref):
    rows_i = lax.broadcasted_iota(jnp.int32, (64, 1), 0)
    pos = rows_i.astype(jnp.float32)
    out_ref[...] = radius_ref[...] * jnp.cos(
        2.0 * np.pi * pos / period_ref[...] + phase_ref[...]
    )


def _pos_bcast_body(radius_ref, period_ref, phase_ref, out_ref):
    pos = lax.broadcasted_iota(jnp.int32, (_S, 1, 1), 0).astype(jnp.float32)
    pe = radius_ref[...] * jnp.cos(
        2.0 * np.pi * pos / period_ref[...] + phase_ref[...]
    )
    out_ref[...] = jnp.broadcast_to(pe, out_ref.shape)


_BB = 128


def _pack_table(token_table):
    tpad = jnp.pad(token_table, ((0, _NPB * 2 * _PB - 1000000), (0, 0)))
    t4 = jnp.swapaxes(tpad.reshape(_NPB, 2, _PB, _D), 1, 2)
    return t4.reshape(_VP, 2 * _D)


def _seq_combine(dst_ref, src_ref, par_ref, si, pos_ref):
    def row_body(r, c):
        h = par_ref[si, pl.ds(r, _L)][0]
        hf = jnp.full((_L,), h, jnp.int32).astype(jnp.float32)
        for c4 in range(_D // _L):
            sl = pl.ds(c4 * _L, _L)
            sh = pl.ds(_D + c4 * _L, _L)
            lo = src_ref[r, sl]
            hi = src_ref[r, sh]
            dst_ref[r, sl] = lo + hf * (hi - lo) + pos_ref[r, sl]
        return c

    lax.fori_loop(0, _S, row_body, 0)


def _sc_gather_add(
    pairs_hbm, par_hbm, table_hbm, pos_hbm, x_hbm,
    idx_v, par_v, pos_v, r0, r1, w0, w1, gs0, gs1, ws0, ws1,
):
    wid = lax.axis_index("s") * _NC + lax.axis_index("c")
    sbase = wid * _SPW
    pltpu.sync_copy(pairs_hbm.at[pl.ds(sbase, _SPW)], idx_v)
    pltpu.sync_copy(par_hbm.at[pl.ds(sbase, _SPW)], par_v)
    pltpu.sync_copy(pos_hbm, pos_v)

    pltpu.async_copy(table_hbm.at[idx_v.at[0, pl.ds(0, _S)]], r0.at[pl.ds(0, _S)], gs0)
    pltpu.async_copy(table_hbm.at[idx_v.at[1, pl.ds(0, _S)]], r1.at[pl.ds(0, _S)], gs1)

    def body(g, carry):
        for slot, (rb, wb, gs, ws) in enumerate(
            ((r0, w0, gs0, ws0), (r1, w1, gs1, ws1))
        ):
            si = 2 * g + slot
            pltpu.make_async_copy(
                table_hbm.at[idx_v.at[si, pl.ds(0, _S)]], rb.at[pl.ds(0, _S)], gs
            ).wait()
            _seq_combine(wb, rb, par_v, si, pos_v)
            @pl.when(si + 2 < _SPW)
            def _():
                pltpu.async_copy(
                    table_hbm.at[idx_v.at[si + 2, pl.ds(0, _S)]],
                    rb.at[pl.ds(0, _S)], gs,
                )
            @pl.when(si >= 2)
            def _():
                pltpu.make_async_copy(
                    wb.at[pl.ds(0, _S)], x_hbm.at[sbase + si], ws
                ).wait()
            pltpu.async_copy(wb.at[pl.ds(0, _S)], x_hbm.at[sbase + si], ws)
        return carry

    lax.fori_loop(0, _SPW // 2, body, 0)
    pltpu.make_async_copy(w0.at[pl.ds(0, _S)], x_hbm.at[sbase], ws0).wait()
    pltpu.make_async_copy(w1.at[pl.ds(0, _S)], x_hbm.at[sbase], ws1).wait()


@functools.cache
def _build_gather_add():
    sc_mesh = plsc.VectorSubcoreMesh(
        core_axis_name="c", subcore_axis_name="s", num_cores=_NC, num_subcores=_NS
    )
    return pl.kernel(
        _sc_gather_add,
        out_type=jax.ShapeDtypeStruct((_B, _S, _D), jnp.float32),
        mesh=sc_mesh,
        scratch_types=[
            pltpu.VMEM((_SPW, 128), jnp.int32),
            pltpu.VMEM((_SPW, 128), jnp.int32),
            pltpu.VMEM((64, 128), jnp.float32),
            pltpu.VMEM((56, 128), jnp.float32),
            pltpu.VMEM((56, 128), jnp.float32),
            pltpu.VMEM((56, _D), jnp.float32),
            pltpu.VMEM((56, _D), jnp.float32),
            pltpu.SemaphoreType.DMA,
            pltpu.SemaphoreType.DMA,
            pltpu.SemaphoreType.DMA,
            pltpu.SemaphoreType.DMA,
        ],
        compiler_params=pltpu.CompilerParams(use_tc_tiling_on_sc=True),
    )


_pos_rep = pl.pallas_call(
    _pos_rep_body,
    out_shape=jax.ShapeDtypeStruct((64, 128), jnp.float32),
)

_BBT = 1024

_pos_bcast = pl.pallas_call(
    _pos_bcast_body,
    grid=(_B // _BBT,),
    in_specs=[
        pl.BlockSpec((1, _D, 1), lambda i: (0, 0, 0)),
        pl.BlockSpec((1, _D, 1), lambda i: (0, 0, 0)),
        pl.BlockSpec((1, _D, 1), lambda i: (0, 0, 0)),
    ],
    out_specs=pl.BlockSpec((_S, _D, _BBT), lambda i: (0, 0, i)),
    out_shape=jax.ShapeDtypeStruct((_S, _D, _B), jnp.float32),
)


def kernel(sequence, token_table, init_radius, period, init_phase):
    seq = sequence.astype(jnp.int32)
    pairs = jnp.pad(((seq >> 11) << 10) + (seq & 1023), ((0, 0), (0, 128 - _S)))
    par = jnp.pad((seq >> 10) & 1, ((0, 0), (0, 128 - _S)))
    table2 = _pack_table(token_table)
    r2 = jnp.pad(init_radius.reshape(1, _D), ((0, 0), (0, 64)))
    p2 = jnp.pad(period.reshape(1, _D), ((0, 0), (0, 64)), constant_values=1.0)
    f2 = jnp.pad(init_phase.reshape(1, _D), ((0, 0), (0, 64)))
    pos = _pos_rep(r2, p2, f2)
    x = _build_gather_add()(pairs, par, table2, pos)
    pb = _pos_bcast(
        init_radius.reshape(1, _D, 1),
        period.reshape(1, _D, 1),
        init_phase.reshape(1, _D, 1),
    )
    position_e = jnp.transpose(pb, (2, 0, 1))
    return (x, init_radius, period, init_phase, position_e)


# speedup vs baseline: 3.4858x; 3.4858x over previous
import functools

import jax
import jax.numpy as jnp
import numpy as np
from jax import lax
from jax.experimental import pallas as pl
from jax.experimental.pallas import tpu as pltpu
from jax.experimental.pallas import tpu_sc as plsc

_B = 4096
_S = 50
_D = 64

_NC = 2
_NS = 16
_NW = _NC * _NS
_L = 16

_SPW = _B // _NW
_PB = 2048
_NPB = 245
_VP = _NPB * _PB


def _pos_rep_body(radius_ref, period_ref, phase_ref, out_ref):
    rows_i = lax.broadcasted_iota(jnp.int32, (64, 1), 0)
    pos = rows_i.astype(jnp.float32)
    out_ref[...] = radius_ref[...] * jnp.cos(
        2.0 * np.pi * pos / period_ref[...] + phase_ref[...]
    )


def _pos_bcast_body(radius_ref, period_ref, phase_ref, out_ref):
    pos = lax.broadcasted_iota(jnp.int32, (_S, 1, 1), 0).astype(jnp.float32)
    pe = radius_ref[...] * jnp.cos(
        2.0 * np.pi * pos / period_ref[...] + phase_ref[...]
    )
    out_ref[...] = jnp.broadcast_to(pe, out_ref.shape)


_BB = 128


def _pack_body(tt_ref, eye_ref, out_ref):
    t = jax.lax.dot_general(
        tt_ref[...], eye_ref[...], (((0,), (0,)), ((), ())),
        preferred_element_type=jnp.float32,
    )
    out_ref[...] = jnp.concatenate([t[:_PB], t[_PB:]], axis=1)


_pack_table = pl.pallas_call(
    _pack_body,
    grid=(_NPB,),
    in_specs=[
        pl.BlockSpec((_D, 2 * _PB), lambda i: (0, i)),
        pl.BlockSpec((_D, _D), lambda i: (0, 0)),
    ],
    out_specs=pl.BlockSpec((_PB, 128), lambda i: (i, 0)),
    out_shape=jax.ShapeDtypeStruct((_VP, 128), jnp.float32),
)


def _seq_combine(dst_ref, src_ref, par_ref, si, pos_ref):
    def row_body(r, c):
        h = par_ref[si, pl.ds(r, _L)][0]
        hf = jnp.full((_L,), h, jnp.int32).astype(jnp.float32)
        for c4 in range(_D // _L):
            sl = pl.ds(c4 * _L, _L)
            sh = pl.ds(_D + c4 * _L, _L)
            lo = src_ref[r, sl]
            hi = src_ref[r, sh]
            dst_ref[r, sl] = lo + hf * (hi - lo) + pos_ref[r, sl]
        return c

    lax.fori_loop(0, _S, row_body, 0)


def _sc_gather_add(
    pairs_hbm, par_hbm, table_hbm, pos_hbm, x_hbm,
    idx_v, par_v, pos_v, r0, r1, w0, w1, gs0, gs1, ws0, ws1,
):
    wid = lax.axis_index("s") * _NC + lax.axis_index("c")
    sbase = wid * _SPW
    pltpu.sync_copy(pairs_hbm.at[pl.ds(sbase, _SPW)], idx_v)
    pltpu.sync_copy(par_hbm.at[pl.ds(sbase, _SPW)], par_v)
    pltpu.sync_copy(pos_hbm, pos_v)

    pltpu.async_copy(table_hbm.at[idx_v.at[0, pl.ds(0, _S)]], r0.at[pl.ds(0, _S)], gs0)
    pltpu.async_copy(table_hbm.at[idx_v.at[1, pl.ds(0, _S)]], r1.at[pl.ds(0, _S)], gs1)

    def body(g, carry):
        for slot, (rb, wb, gs, ws) in enumerate(
            ((r0, w0, gs0, ws0), (r1, w1, gs1, ws1))
        ):
            si = 2 * g + slot
            pltpu.make_async_copy(
                table_hbm.at[idx_v.at[si, pl.ds(0, _S)]], rb.at[pl.ds(0, _S)], gs
            ).wait()
            _seq_combine(wb, rb, par_v, si, pos_v)
            @pl.when(si + 2 < _SPW)
            def _():
                pltpu.async_copy(
                    table_hbm.at[idx_v.at[si + 2, pl.ds(0, _S)]],
                    rb.at[pl.ds(0, _S)], gs,
                )
            @pl.when(si >= 2)
            def _():
                pltpu.make_async_copy(
                    wb.at[pl.ds(0, _S)], x_hbm.at[sbase + si], ws
                ).wait()
            pltpu.async_copy(wb.at[pl.ds(0, _S)], x_hbm.at[sbase + si], ws)
        return carry

    lax.fori_loop(0, _SPW // 2, body, 0)
    pltpu.make_async_copy(w0.at[pl.ds(0, _S)], x_hbm.at[sbase], ws0).wait()
    pltpu.make_async_copy(w1.at[pl.ds(0, _S)], x_hbm.at[sbase], ws1).wait()


@functools.cache
def _build_gather_add():
    sc_mesh = plsc.VectorSubcoreMesh(
        core_axis_name="c", subcore_axis_name="s", num_cores=_NC, num_subcores=_NS
    )
    return pl.kernel(
        _sc_gather_add,
        out_type=jax.ShapeDtypeStruct((_B, _S, _D), jnp.float32),
        mesh=sc_mesh,
        scratch_types=[
            pltpu.VMEM((_SPW, 128), jnp.int32),
            pltpu.VMEM((_SPW, 128), jnp.int32),
            pltpu.VMEM((64, 128), jnp.float32),
            pltpu.VMEM((56, 128), jnp.float32),
            pltpu.VMEM((56, 128), jnp.float32),
            pltpu.VMEM((56, _D), jnp.float32),
            pltpu.VMEM((56, _D), jnp.float32),
            pltpu.SemaphoreType.DMA,
            pltpu.SemaphoreType.DMA,
            pltpu.SemaphoreType.DMA,
            pltpu.SemaphoreType.DMA,
        ],
        compiler_params=pltpu.CompilerParams(use_tc_tiling_on_sc=True),
    )


_pos_rep = pl.pallas_call(
    _pos_rep_body,
    out_shape=jax.ShapeDtypeStruct((64, 128), jnp.float32),
)

_BBT = 1024

_pos_bcast = pl.pallas_call(
    _pos_bcast_body,
    grid=(_B // _BBT,),
    in_specs=[
        pl.BlockSpec((1, _D, 1), lambda i: (0, 0, 0)),
        pl.BlockSpec((1, _D, 1), lambda i: (0, 0, 0)),
        pl.BlockSpec((1, _D, 1), lambda i: (0, 0, 0)),
    ],
    out_specs=pl.BlockSpec((_S, _D, _BBT), lambda i: (0, 0, i)),
    out_shape=jax.ShapeDtypeStruct((_S, _D, _B), jnp.float32),
)


def kernel(sequence, token_table, init_radius, period, init_phase):
    seq = sequence.astype(jnp.int32)
    pairs = jnp.pad(((seq >> 12) << 11) + (seq & 2047), ((0, 0), (0, 128 - _S)))
    par = jnp.pad((seq >> 11) & 1, ((0, 0), (0, 128 - _S)))
    table2 = _pack_table(token_table.T, jnp.eye(_D, dtype=jnp.float32))
    r2 = jnp.pad(init_radius.reshape(1, _D), ((0, 0), (0, 64)))
    p2 = jnp.pad(period.reshape(1, _D), ((0, 0), (0, 64)), constant_values=1.0)
    f2 = jnp.pad(init_phase.reshape(1, _D), ((0, 0), (0, 64)))
    pos = _pos_rep(r2, p2, f2)
    x = _build_gather_add()(pairs, par, table2, pos)
    pb = _pos_bcast(
        init_radius.reshape(1, _D, 1),
        period.reshape(1, _D, 1),
        init_phase.reshape(1, _D, 1),
    )
    position_e = jnp.transpose(pb, (2, 0, 1))
    return (x, init_radius, period, init_phase, position_e)


# speedup vs baseline: 3.9769x; 1.1409x over previous
import functools

import jax
import jax.numpy as jnp
import numpy as np
from jax import lax
from jax.experimental import pallas as pl
from jax.experimental.pallas import tpu as pltpu
from jax.experimental.pallas import tpu_sc as plsc

_B = 4096
_S = 50
_D = 64

_NC = 2
_NS = 16
_NW = _NC * _NS
_L = 16

_SPW = _B // _NW
_PB = 4096
_NPB = 123
_VP = _NPB * _PB


def _pos_rep_body(radius_ref, period_ref, phase_ref, out_ref):
    rows_i = lax.broadcasted_iota(jnp.int32, (64, 1), 0)
    pos = rows_i.astype(jnp.float32)
    out_ref[...] = radius_ref[...] * jnp.cos(
        2.0 * np.pi * pos / period_ref[...] + phase_ref[...]
    )


def _pos_bcast_body(radius_ref, period_ref, phase_ref, out_ref):
    pos = lax.broadcasted_iota(jnp.int32, (_S, 1, 1), 0).astype(jnp.float32)
    pe = radius_ref[...] * jnp.cos(
        2.0 * np.pi * pos / period_ref[...] + phase_ref[...]
    )
    out_ref[...] = jnp.broadcast_to(pe, out_ref.shape)


_BB = 128


def _pack_body(tt_ref, eye_ref, out_ref):
    t = jax.lax.dot_general(
        tt_ref[...], eye_ref[...], (((0,), (0,)), ((), ())),
        preferred_element_type=jnp.float32,
    )
    out_ref[...] = jnp.concatenate([t[:_PB], t[_PB:]], axis=1)


_pack_table = pl.pallas_call(
    _pack_body,
    grid=(_NPB,),
    in_specs=[
        pl.BlockSpec((_D, 2 * _PB), lambda i: (0, i)),
        pl.BlockSpec((_D, _D), lambda i: (0, 0)),
    ],
    out_specs=pl.BlockSpec((_PB, 128), lambda i: (i, 0)),
    out_shape=jax.ShapeDtypeStruct((_VP, 128), jnp.float32),
)


def _seq_combine(dst_ref, src_ref, par_ref, si, pos_ref):
    def row_body(r, c):
        h = par_ref[si, pl.ds(r, _L)][0]
        hf = jnp.full((_L,), h, jnp.int32).astype(jnp.float32)
        for c4 in range(_D // _L):
            sl = pl.ds(c4 * _L, _L)
            sh = pl.ds(_D + c4 * _L, _L)
            lo = src_ref[r, sl]
            hi = src_ref[r, sh]
            dst_ref[r, sl] = lo + hf * (hi - lo) + pos_ref[r, sl]
        return c

    lax.fori_loop(0, _S, row_body, 0)


def _sc_gather_add(
    pairs_hbm, par_hbm, table_hbm, pos_hbm, x_hbm,
    idx_v, par_v, pos_v, r0, r1, w0, w1, gs0, gs1, ws0, ws1,
):
    wid = lax.axis_index("s") * _NC + lax.axis_index("c")
    sbase = wid * _SPW
    pltpu.sync_copy(pairs_hbm.at[pl.ds(sbase, _SPW)], idx_v)
    pltpu.sync_copy(par_hbm.at[pl.ds(sbase, _SPW)], par_v)
    pltpu.sync_copy(pos_hbm, pos_v)

    pltpu.async_copy(table_hbm.at[idx_v.at[0, pl.ds(0, _S)]], r0.at[pl.ds(0, _S)], gs0)
    pltpu.async_copy(table_hbm.at[idx_v.at[1, pl.ds(0, _S)]], r1.at[pl.ds(0, _S)], gs1)

    def body(g, carry):
        for slot, (rb, wb, gs, ws) in enumerate(
            ((r0, w0, gs0, ws0), (r1, w1, gs1, ws1))
        ):
            si = 2 * g + slot
            pltpu.make_async_copy(
                table_hbm.at[idx_v.at[si, pl.ds(0, _S)]], rb.at[pl.ds(0, _S)], gs
            ).wait()
            _seq_combine(wb, rb, par_v, si, pos_v)
            @pl.when(si + 2 < _SPW)
            def _():
                pltpu.async_copy(
                    table_hbm.at[idx_v.at[si + 2, pl.ds(0, _S)]],
                    rb.at[pl.ds(0, _S)], gs,
                )
            @pl.when(si >= 2)
            def _():
                pltpu.make_async_copy(
                    wb.at[pl.ds(0, _S)], x_hbm.at[sbase + si], ws
                ).wait()
            pltpu.async_copy(wb.at[pl.ds(0, _S)], x_hbm.at[sbase + si], ws)
        return carry

    lax.fori_loop(0, _SPW // 2, body, 0)
    pltpu.make_async_copy(w0.at[pl.ds(0, _S)], x_hbm.at[sbase], ws0).wait()
    pltpu.make_async_copy(w1.at[pl.ds(0, _S)], x_hbm.at[sbase], ws1).wait()


@functools.cache
def _build_gather_add():
    sc_mesh = plsc.VectorSubcoreMesh(
        core_axis_name="c", subcore_axis_name="s", num_cores=_NC, num_subcores=_NS
    )
    return pl.kernel(
        _sc_gather_add,
        out_type=jax.ShapeDtypeStruct((_B, _S, _D), jnp.float32),
        mesh=sc_mesh,
        scratch_types=[
            pltpu.VMEM((_SPW, 128), jnp.int32),
            pltpu.VMEM((_SPW, 128), jnp.int32),
            pltpu.VMEM((64, 128), jnp.float32),
            pltpu.VMEM((56, 128), jnp.float32),
            pltpu.VMEM((56, 128), jnp.float32),
            pltpu.VMEM((56, _D), jnp.float32),
            pltpu.VMEM((56, _D), jnp.float32),
            pltpu.SemaphoreType.DMA,
            pltpu.SemaphoreType.DMA,
            pltpu.SemaphoreType.DMA,
            pltpu.SemaphoreType.DMA,
        ],
        compiler_params=pltpu.CompilerParams(use_tc_tiling_on_sc=True),
    )


_pos_rep = pl.pallas_call(
    _pos_rep_body,
    out_shape=jax.ShapeDtypeStruct((64, 128), jnp.float32),
)

_BBT = 1024

_pos_bcast = pl.pallas_call(
    _pos_bcast_body,
    grid=(_B // _BBT,),
    in_specs=[
        pl.BlockSpec((1, _D, 1), lambda i: (0, 0, 0)),
        pl.BlockSpec((1, _D, 1), lambda i: (0, 0, 0)),
        pl.BlockSpec((1, _D, 1), lambda i: (0, 0, 0)),
    ],
    out_specs=pl.BlockSpec((_S, _D, _BBT), lambda i: (0, 0, i)),
    out_shape=jax.ShapeDtypeStruct((_S, _D, _B), jnp.float32),
)


def kernel(sequence, token_table, init_radius, period, init_phase):
    seq = sequence.astype(jnp.int32)
    pairs = jnp.pad(((seq >> 13) << 12) + (seq & 4095), ((0, 0), (0, 128 - _S)))
    par = jnp.pad((seq >> 12) & 1, ((0, 0), (0, 128 - _S)))
    table2 = _pack_table(token_table.T, jnp.eye(_D, dtype=jnp.float32))
    r2 = jnp.pad(init_radius.reshape(1, _D), ((0, 0), (0, 64)))
    p2 = jnp.pad(period.reshape(1, _D), ((0, 0), (0, 64)), constant_values=1.0)
    f2 = jnp.pad(init_phase.reshape(1, _D), ((0, 0), (0, 64)))
    pos = _pos_rep(r2, p2, f2)
    x = _build_gather_add()(pairs, par, table2, pos)
    pb = _pos_bcast(
        init_radius.reshape(1, _D, 1),
        period.reshape(1, _D, 1),
        init_phase.reshape(1, _D, 1),
    )
    position_e = jnp.transpose(pb, (2, 0, 1))
    return (x, init_radius, period, init_phase, position_e)


# speedup vs baseline: 4.2748x; 1.0749x over previous
import functools

import jax
import jax.numpy as jnp
import numpy as np
from jax import lax
from jax.experimental import pallas as pl
from jax.experimental.pallas import tpu as pltpu
from jax.experimental.pallas import tpu_sc as plsc

_B = 4096
_S = 50
_D = 64

_NC = 2
_NS = 16
_NW = _NC * _NS
_L = 16

_SPW = _B // _NW
_PB = 8192
_NPB = 62
_VP = _NPB * _PB


def _pos_rep_body(radius_ref, period_ref, phase_ref, out_ref):
    rows_i = lax.broadcasted_iota(jnp.int32, (64, 1), 0)
    pos = rows_i.astype(jnp.float32)
    out_ref[...] = radius_ref[...] * jnp.cos(
        2.0 * np.pi * pos / period_ref[...] + phase_ref[...]
    )


def _pos_bcast_body(radius_ref, period_ref, phase_ref, out_ref):
    pos = lax.broadcasted_iota(jnp.int32, (_S, 1, 1), 0).astype(jnp.float32)
    pe = radius_ref[...] * jnp.cos(
        2.0 * np.pi * pos / period_ref[...] + phase_ref[...]
    )
    out_ref[...] = jnp.broadcast_to(pe, out_ref.shape)


_BB = 128


def _pack_body(tt_ref, eye_ref, out_ref):
    t = jax.lax.dot_general(
        tt_ref[...], eye_ref[...], (((0,), (0,)), ((), ())),
        preferred_element_type=jnp.float32,
    )
    out_ref[...] = jnp.concatenate([t[:_PB], t[_PB:]], axis=1)


_pack_table = pl.pallas_call(
    _pack_body,
    grid=(_NPB,),
    in_specs=[
        pl.BlockSpec((_D, 2 * _PB), lambda i: (0, i)),
        pl.BlockSpec((_D, _D), lambda i: (0, 0)),
    ],
    out_specs=pl.BlockSpec((_PB, 128), lambda i: (i, 0)),
    out_shape=jax.ShapeDtypeStruct((_VP, 128), jnp.float32),
)


def _seq_combine(dst_ref, src_ref, par_ref, si, pos_ref):
    def row_body(r, c):
        h = par_ref[si, pl.ds(r, _L)][0]
        hf = jnp.full((_L,), h, jnp.int32).astype(jnp.float32)
        for c4 in range(_D // _L):
            sl = pl.ds(c4 * _L, _L)
            sh = pl.ds(_D + c4 * _L, _L)
            lo = src_ref[r, sl]
            hi = src_ref[r, sh]
            dst_ref[r, sl] = lo + hf * (hi - lo) + pos_ref[r, sl]
        return c

    lax.fori_loop(0, _S, row_body, 0)


def _sc_gather_add(
    pairs_hbm, par_hbm, table_hbm, pos_hbm, x_hbm,
    idx_v, par_v, pos_v, r0, r1, w0, w1, gs0, gs1, ws0, ws1,
):
    wid = lax.axis_index("s") * _NC + lax.axis_index("c")
    sbase = wid * _SPW
    pltpu.sync_copy(pairs_hbm.at[pl.ds(sbase, _SPW)], idx_v)
    pltpu.sync_copy(par_hbm.at[pl.ds(sbase, _SPW)], par_v)
    pltpu.sync_copy(pos_hbm, pos_v)

    pltpu.async_copy(table_hbm.at[idx_v.at[0, pl.ds(0, _S)]], r0.at[pl.ds(0, _S)], gs0)
    pltpu.async_copy(table_hbm.at[idx_v.at[1, pl.ds(0, _S)]], r1.at[pl.ds(0, _S)], gs1)

    def body(g, carry):
        for slot, (rb, wb, gs, ws) in enumerate(
            ((r0, w0, gs0, ws0), (r1, w1, gs1, ws1))
        ):
            si = 2 * g + slot
            pltpu.make_async_copy(
                table_hbm.at[idx_v.at[si, pl.ds(0, _S)]], rb.at[pl.ds(0, _S)], gs
            ).wait()
            _seq_combine(wb, rb, par_v, si, pos_v)
            @pl.when(si + 2 < _SPW)
            def _():
                pltpu.async_copy(
                    table_hbm.at[idx_v.at[si + 2, pl.ds(0, _S)]],
                    rb.at[pl.ds(0, _S)], gs,
                )
            @pl.when(si >= 2)
            def _():
                pltpu.make_async_copy(
                    wb.at[pl.ds(0, _S)], x_hbm.at[sbase + si], ws
                ).wait()
            pltpu.async_copy(wb.at[pl.ds(0, _S)], x_hbm.at[sbase + si], ws)
        return carry

    lax.fori_loop(0, _SPW // 2, body, 0)
    pltpu.make_async_copy(w0.at[pl.ds(0, _S)], x_hbm.at[sbase], ws0).wait()
    pltpu.make_async_copy(w1.at[pl.ds(0, _S)], x_hbm.at[sbase], ws1).wait()


@functools.cache
def _build_gather_add():
    sc_mesh = plsc.VectorSubcoreMesh(
        core_axis_name="c", subcore_axis_name="s", num_cores=_NC, num_subcores=_NS
    )
    return pl.kernel(
        _sc_gather_add,
        out_type=jax.ShapeDtypeStruct((_B, _S, _D), jnp.float32),
        mesh=sc_mesh,
        scratch_types=[
            pltpu.VMEM((_SPW, 128), jnp.int32),
            pltpu.VMEM((_SPW, 128), jnp.int32),
            pltpu.VMEM((64, 128), jnp.float32),
            pltpu.VMEM((56, 128), jnp.float32),
            pltpu.VMEM((56, 128), jnp.float32),
            pltpu.VMEM((56, _D), jnp.float32),
            pltpu.VMEM((56, _D), jnp.float32),
            pltpu.SemaphoreType.DMA,
            pltpu.SemaphoreType.DMA,
            pltpu.SemaphoreType.DMA,
            pltpu.SemaphoreType.DMA,
        ],
        compiler_params=pltpu.CompilerParams(use_tc_tiling_on_sc=True),
    )


_pos_rep = pl.pallas_call(
    _pos_rep_body,
    out_shape=jax.ShapeDtypeStruct((64, 128), jnp.float32),
)

_BBT = 1024

_pos_bcast = pl.pallas_call(
    _pos_bcast_body,
    grid=(_B // _BBT,),
    in_specs=[
        pl.BlockSpec((1, _D, 1), lambda i: (0, 0, 0)),
        pl.BlockSpec((1, _D, 1), lambda i: (0, 0, 0)),
        pl.BlockSpec((1, _D, 1), lambda i: (0, 0, 0)),
    ],
    out_specs=pl.BlockSpec((_S, _D, _BBT), lambda i: (0, 0, i)),
    out_shape=jax.ShapeDtypeStruct((_S, _D, _B), jnp.float32),
)


def kernel(sequence, token_table, init_radius, period, init_phase):
    seq = sequence.astype(jnp.int32)
    pairs = jnp.pad(((seq >> 14) << 13) + (seq & 8191), ((0, 0), (0, 128 - _S)))
    par = jnp.pad((seq >> 13) & 1, ((0, 0), (0, 128 - _S)))
    table2 = _pack_table(token_table.T, jnp.eye(_D, dtype=jnp.float32))
    r2 = jnp.pad(init_radius.reshape(1, _D), ((0, 0), (0, 64)))
    p2 = jnp.pad(period.reshape(1, _D), ((0, 0), (0, 64)), constant_values=1.0)
    f2 = jnp.pad(init_phase.reshape(1, _D), ((0, 0), (0, 64)))
    pos = _pos_rep(r2, p2, f2)
    x = _build_gather_add()(pairs, par, table2, pos)
    pb = _pos_bcast(
        init_radius.reshape(1, _D, 1),
        period.reshape(1, _D, 1),
        init_phase.reshape(1, _D, 1),
    )
    position_e = jnp.transpose(pb, (2, 0, 1))
    return (x, init_radius, period, init_phase, position_e)
